# Initial kernel scaffold; baseline (speedup 1.0000x reference)
import jax
import jax.numpy as jnp
from jax.experimental import pallas as pl


def kernel(inputs, table, W1, b1, W2, b2):
    raise NotImplementedError("write your pallas kernel here")


# speedup vs baseline: 52.8283x; 52.8283x over previous
import functools

import jax
import jax.numpy as jnp
from jax import lax
from jax.experimental import pallas as pl
from jax.experimental.pallas import tpu as pltpu
from jax.experimental.pallas import tpu_sc as plsc

_VOCAB = 10000
_EMBED = 16
_BATCH = 16384
_MAXLEN = 500

_NC = 2
_NS = 16
_NW = _NC * _NS
_ROWS_PER_W = _BATCH // _NW
_C = 8
_H = 4
_G = 100
_GPR = _MAXLEN // _G
_GPH = _H * _GPR
_NCHUNK = _ROWS_PER_W // _C


def _sc_pool(idx_hbm, table_hbm, sums_hbm, idx_v, rows_v, acc_v, sem):
    wid = lax.axis_index("s") * _NC + lax.axis_index("c")
    row0 = wid * _ROWS_PER_W

    def chunk_body(ci, carry):
        r0 = row0 + ci * _C
        pltpu.sync_copy(idx_hbm.at[pl.ds(r0 * _GPR, _C * _GPR)], idx_v)
        for h in range(_C // _H):
            copies = []
            for g in range(_GPH):
                copies.append(
                    pltpu.async_copy(
                        table_hbm.at[idx_v.at[h * _GPH + g]], rows_v.at[g], sem
                    )
                )
            for c in copies:
                c.wait()
            for r in range(_H):
                def red(j, accs):
                    a0, a1 = accs
                    base = r * _GPR
                    for g in range(_GPR):
                        a0 = a0 + rows_v[base + g, 2 * j, :]
                        a1 = a1 + rows_v[base + g, 2 * j + 1, :]
                    return (a0, a1)

                zero = jnp.zeros((_EMBED,), jnp.float32)
                a0, a1 = lax.fori_loop(0, _G // 2, red, (zero, zero))
                acc_v[h * _H + r, :] = a0 + a1
        pltpu.sync_copy(acc_v, sums_hbm.at[pl.ds(r0, _C)])
        return carry

    lax.fori_loop(0, _NCHUNK, chunk_body, 0)


def _make_sc_pool():
    mesh = plsc.VectorSubcoreMesh(core_axis_name="c", subcore_axis_name="s")
    return functools.partial(
        pl.kernel,
        mesh=mesh,
        compiler_params=pltpu.CompilerParams(use_tc_tiling_on_sc=False),
        out_type=jax.ShapeDtypeStruct((_BATCH, _EMBED), jnp.float32),
        scratch_types=[
            pltpu.VMEM((_C * _GPR, _G), jnp.int32),
            pltpu.VMEM((_GPH, _G, _EMBED), jnp.float32),
            pltpu.VMEM((_C, _EMBED), jnp.float32),
            pltpu.SemaphoreType.DMA,
        ],
    )(_sc_pool)


def _mlp_body(sums_ref, w1_ref, b1_ref, w2r_ref, b2_ref, out_ref):
    pooled = sums_ref[...] * (1.0 / _MAXLEN)
    h = jnp.dot(pooled, w1_ref[...], preferred_element_type=jnp.float32)
    h = jnp.maximum(h + b1_ref[...], 0.0)
    z = jnp.sum(h * w2r_ref[...], axis=1, keepdims=True) + b2_ref[...]
    out_ref[...] = 1.0 / (1.0 + jnp.exp(-z))


def _mlp(sums, W1, b1, W2, b2):
    blk = 2048
    grid = (_BATCH // blk,)
    return pl.pallas_call(
        _mlp_body,
        grid=grid,
        in_specs=[
            pl.BlockSpec((blk, _EMBED), lambda i: (i, 0)),
            pl.BlockSpec((_EMBED, _EMBED), lambda i: (0, 0)),
            pl.BlockSpec((1, _EMBED), lambda i: (0, 0)),
            pl.BlockSpec((1, _EMBED), lambda i: (0, 0)),
            pl.BlockSpec((1, 1), lambda i: (0, 0)),
        ],
        out_specs=pl.BlockSpec((blk, 1), lambda i: (i, 0)),
        out_shape=jax.ShapeDtypeStruct((_BATCH, 1), jnp.float32),
    )(sums, W1, b1.reshape(1, _EMBED), W2.reshape(1, _EMBED), b2.reshape(1, 1))


def kernel(inputs, table, W1, b1, W2, b2):
    idx = inputs.astype(jnp.int32).reshape(_BATCH * _GPR, _G)
    sums = _make_sc_pool()(idx, table)
    return _mlp(sums, W1, b1, W2, b2)


# speedup vs baseline: 74.1340x; 1.4033x over previous
import functools

import jax
import jax.numpy as jnp
from jax import lax
from jax.experimental import pallas as pl
from jax.experimental.pallas import tpu as pltpu
from jax.experimental.pallas import tpu_sc as plsc

_VOCAB = 10000
_EMBED = 16
_BATCH = 16384
_MAXLEN = 500

_NC = 2
_NS = 16
_NW = _NC * _NS
_ROWS_PER_W = _BATCH // _NW
_C = 8
_H = 4
_G = 100
_GPR = _MAXLEN // _G
_GPH = _H * _GPR
_NCHUNK = _ROWS_PER_W // _C


def _sc_pool(idx_hbm, table_hbm, sums_hbm, idx_v, rows_a, rows_b, acc_v,
             sem_a, sem_b):
    wid = lax.axis_index("s") * _NC + lax.axis_index("c")
    row0 = wid * _ROWS_PER_W

    def fire(gbase, rows, sem):
        for g in range(_GPH):
            pltpu.async_copy(table_hbm.at[idx_v.at[gbase + g]], rows.at[g], sem)

    def drain(gbase, rows, sem):
        for g in range(_GPH):
            pltpu.make_async_copy(
                table_hbm.at[idx_v.at[gbase + g]], rows.at[g], sem
            ).wait()

    def reduce_half(rows, half):
        for r in range(_H):
            base = r * _GPR

            def red(j, accs):
                out = list(accs)
                for g in range(_GPR):
                    for u in range(4):
                        out[u] = out[u] + rows[base + g, 4 * j + u, :]
                return tuple(out)

            zero = jnp.zeros((_EMBED,), jnp.float32)
            a = lax.fori_loop(0, _G // 4, red, (zero,) * 4)
            acc_v[half * _H + r, :] = (a[0] + a[1]) + (a[2] + a[3])

    pltpu.sync_copy(idx_hbm.at[pl.ds(row0 * _GPR, _C * _GPR)], idx_v)
    fire(0, rows_a, sem_a)

    def chunk_body(ci, carry):
        r0 = row0 + ci * _C
        fire(_GPH, rows_b, sem_b)
        drain(0, rows_a, sem_a)
        reduce_half(rows_a, 0)
        drain(_GPH, rows_b, sem_b)

        @pl.when(ci < _NCHUNK - 1)
        def _():
            r0n = row0 + (ci + 1) * _C
            pltpu.sync_copy(idx_hbm.at[pl.ds(r0n * _GPR, _C * _GPR)], idx_v)
            fire(0, rows_a, sem_a)

        reduce_half(rows_b, 1)
        pltpu.sync_copy(acc_v, sums_hbm.at[pl.ds(r0, _C)])
        return carry

    lax.fori_loop(0, _NCHUNK, chunk_body, 0)


def _make_sc_pool():
    mesh = plsc.VectorSubcoreMesh(core_axis_name="c", subcore_axis_name="s")
    return functools.partial(
        pl.kernel,
        mesh=mesh,
        compiler_params=pltpu.CompilerParams(use_tc_tiling_on_sc=False),
        out_type=jax.ShapeDtypeStruct((_BATCH, _EMBED), jnp.float32),
        scratch_types=[
            pltpu.VMEM((_C * _GPR, _G), jnp.int32),
            pltpu.VMEM((_GPH, _G, _EMBED), jnp.float32),
            pltpu.VMEM((_GPH, _G, _EMBED), jnp.float32),
            pltpu.VMEM((_C, _EMBED), jnp.float32),
            pltpu.SemaphoreType.DMA,
            pltpu.SemaphoreType.DMA,
        ],
    )(_sc_pool)


def _mlp_body(sums_ref, w1_ref, b1_ref, w2r_ref, b2_ref, out_ref):
    pooled = sums_ref[...] * (1.0 / _MAXLEN)
    h = jnp.dot(pooled, w1_ref[...], preferred_element_type=jnp.float32)
    h = jnp.maximum(h + b1_ref[...], 0.0)
    z = jnp.sum(h * w2r_ref[...], axis=1, keepdims=True) + b2_ref[...]
    out_ref[...] = 1.0 / (1.0 + jnp.exp(-z))


def _mlp(sums, W1, b1, W2, b2):
    blk = 2048
    grid = (_BATCH // blk,)
    return pl.pallas_call(
        _mlp_body,
        grid=grid,
        in_specs=[
            pl.BlockSpec((blk, _EMBED), lambda i: (i, 0)),
            pl.BlockSpec((_EMBED, _EMBED), lambda i: (0, 0)),
            pl.BlockSpec((1, _EMBED), lambda i: (0, 0)),
            pl.BlockSpec((1, _EMBED), lambda i: (0, 0)),
            pl.BlockSpec((1, 1), lambda i: (0, 0)),
        ],
        out_specs=pl.BlockSpec((blk, 1), lambda i: (i, 0)),
        out_shape=jax.ShapeDtypeStruct((_BATCH, 1), jnp.float32),
    )(sums, W1, b1.reshape(1, _EMBED), W2.reshape(1, _EMBED), b2.reshape(1, 1))


def kernel(inputs, table, W1, b1, W2, b2):
    idx = inputs.astype(jnp.int32).reshape(_BATCH * _GPR, _G)
    sums = _make_sc_pool()(idx, table)
    return _mlp(sums, W1, b1, W2, b2)


# speedup vs baseline: 80.7088x; 1.0887x over previous
import functools

import jax
import jax.numpy as jnp
from jax import lax
from jax.experimental import pallas as pl
from jax.experimental.pallas import tpu as pltpu
from jax.experimental.pallas import tpu_sc as plsc

_VOCAB = 10000
_EMBED = 16
_BATCH = 16384
_MAXLEN = 500

_NC = 2
_NS = 16
_NW = _NC * _NS
_ROWS_PER_W = _BATCH // _NW
_C = 8
_H = 4
_G = 500
_GPR = _MAXLEN // _G
_GPH = _H * _GPR
_NCHUNK = _ROWS_PER_W // _C


def _sc_pool(idx_hbm, table_hbm, sums_hbm, idx_v, rows_a, rows_b, acc_v,
             sem_a, sem_b):
    wid = lax.axis_index("s") * _NC + lax.axis_index("c")
    row0 = wid * _ROWS_PER_W

    def fire(gbase, rows, sem):
        for g in range(_GPH):
            pltpu.async_copy(table_hbm.at[idx_v.at[gbase + g]], rows.at[g], sem)

    def drain(gbase, rows, sem):
        for g in range(_GPH):
            pltpu.make_async_copy(
                table_hbm.at[idx_v.at[gbase + g]], rows.at[g], sem
            ).wait()

    def reduce_half(rows, half):
        for r in range(_H):
            base = r * _GPR

            def red(j, accs):
                out = list(accs)
                for g in range(_GPR):
                    for u in range(4):
                        out[u] = out[u] + rows[base + g, 4 * j + u, :]
                return tuple(out)

            zero = jnp.zeros((_EMBED,), jnp.float32)
            a = lax.fori_loop(0, _G // 4, red, (zero,) * 4)
            acc_v[half * _H + r, :] = (a[0] + a[1]) + (a[2] + a[3])

    pltpu.sync_copy(idx_hbm.at[pl.ds(row0, _C)], idx_v)
    fire(0, rows_a, sem_a)

    def chunk_body(ci, carry):
        r0 = row0 + ci * _C
        fire(_GPH, rows_b, sem_b)
        drain(0, rows_a, sem_a)
        reduce_half(rows_a, 0)
        drain(_GPH, rows_b, sem_b)

        @pl.when(ci < _NCHUNK - 1)
        def _():
            r0n = row0 + (ci + 1) * _C
            pltpu.sync_copy(idx_hbm.at[pl.ds(r0n, _C)], idx_v)
            fire(0, rows_a, sem_a)

        reduce_half(rows_b, 1)
        pltpu.sync_copy(acc_v, sums_hbm.at[pl.ds(r0, _C)])
        return carry

    lax.fori_loop(0, _NCHUNK, chunk_body, 0)


def _make_sc_pool():
    mesh = plsc.VectorSubcoreMesh(core_axis_name="c", subcore_axis_name="s")
    return functools.partial(
        pl.kernel,
        mesh=mesh,
        compiler_params=pltpu.CompilerParams(use_tc_tiling_on_sc=False),
        out_type=jax.ShapeDtypeStruct((_BATCH, _EMBED), jnp.float32),
        scratch_types=[
            pltpu.VMEM((_C, _G), jnp.int32),
            pltpu.VMEM((_GPH, _G, _EMBED), jnp.float32),
            pltpu.VMEM((_GPH, _G, _EMBED), jnp.float32),
            pltpu.VMEM((_C, _EMBED), jnp.float32),
            pltpu.SemaphoreType.DMA,
            pltpu.SemaphoreType.DMA,
        ],
    )(_sc_pool)


def _mlp_body(sums_ref, w1_ref, b1_ref, w2r_ref, b2_ref, out_ref):
    pooled = sums_ref[...] * (1.0 / _MAXLEN)
    h = jnp.dot(pooled, w1_ref[...], preferred_element_type=jnp.float32)
    h = jnp.maximum(h + b1_ref[...], 0.0)
    z = jnp.sum(h * w2r_ref[...], axis=1, keepdims=True) + b2_ref[...]
    out_ref[...] = 1.0 / (1.0 + jnp.exp(-z))


def _mlp(sums, W1, b1, W2, b2):
    blk = 2048
    grid = (_BATCH // blk,)
    return pl.pallas_call(
        _mlp_body,
        grid=grid,
        in_specs=[
            pl.BlockSpec((blk, _EMBED), lambda i: (i, 0)),
            pl.BlockSpec((_EMBED, _EMBED), lambda i: (0, 0)),
            pl.BlockSpec((1, _EMBED), lambda i: (0, 0)),
            pl.BlockSpec((1, _EMBED), lambda i: (0, 0)),
            pl.BlockSpec((1, 1), lambda i: (0, 0)),
        ],
        out_specs=pl.BlockSpec((blk, 1), lambda i: (i, 0)),
        out_shape=jax.ShapeDtypeStruct((_BATCH, 1), jnp.float32),
    )(sums, W1, b1.reshape(1, _EMBED), W2.reshape(1, _EMBED), b2.reshape(1, 1))


def kernel(inputs, table, W1, b1, W2, b2):
    idx = inputs.astype(jnp.int32)
    sums = _make_sc_pool()(idx, table)
    return _mlp(sums, W1, b1, W2, b2)


# speedup vs baseline: 102.2524x; 1.2669x over previous
import functools

import jax
import jax.numpy as jnp
from jax import lax
from jax.experimental import pallas as pl
from jax.experimental.pallas import tpu as pltpu
from jax.experimental.pallas import tpu_sc as plsc

_VOCAB = 10000
_EMBED = 16
_BATCH = 16384
_MAXLEN = 500

_NC = 2
_NS = 16
_NW = _NC * _NS
_ROWS_PER_W = _BATCH // _NW
_C = 8
_H = 4
_G = 500
_GPR = _MAXLEN // _G
_GPH = _H * _GPR
_NCHUNK = _ROWS_PER_W // _C


def _sc_pool(idx_hbm, table_hbm, sums_hbm, idx_v, rows_a, rows_b, acc_v,
             table_sh, sem_a, sem_b):
    sid = lax.axis_index("s")
    wid = sid * _NC + lax.axis_index("c")
    row0 = wid * _ROWS_PER_W

    @pl.when(sid < 10)
    def _():
        pltpu.sync_copy(table_hbm.at[pl.ds(sid * 1000, 1000)],
                        table_sh.at[pl.ds(sid * 1000, 1000)])
    plsc.subcore_barrier()

    def fire(gbase, rows, sem):
        for g in range(_GPH):
            pltpu.async_copy(table_sh.at[idx_v.at[gbase + g]], rows.at[g], sem)

    def drain(gbase, rows, sem):
        for g in range(_GPH):
            pltpu.make_async_copy(
                table_sh.at[idx_v.at[gbase + g]], rows.at[g], sem
            ).wait()

    def reduce_half(rows, half):
        for r in range(_H):
            base = r * _GPR

            def red(j, accs):
                out = list(accs)
                for g in range(_GPR):
                    for u in range(4):
                        out[u] = out[u] + rows[base + g, 4 * j + u, :]
                return tuple(out)

            zero = jnp.zeros((_EMBED,), jnp.float32)
            a = lax.fori_loop(0, _G // 4, red, (zero,) * 4)
            acc_v[half * _H + r, :] = (a[0] + a[1]) + (a[2] + a[3])

    pltpu.sync_copy(idx_hbm.at[pl.ds(row0, _C)], idx_v)
    fire(0, rows_a, sem_a)

    def chunk_body(ci, carry):
        r0 = row0 + ci * _C
        fire(_GPH, rows_b, sem_b)
        drain(0, rows_a, sem_a)
        reduce_half(rows_a, 0)
        drain(_GPH, rows_b, sem_b)

        @pl.when(ci < _NCHUNK - 1)
        def _():
            r0n = row0 + (ci + 1) * _C
            pltpu.sync_copy(idx_hbm.at[pl.ds(r0n, _C)], idx_v)
            fire(0, rows_a, sem_a)

        reduce_half(rows_b, 1)
        pltpu.sync_copy(acc_v, sums_hbm.at[pl.ds(r0, _C)])
        return carry

    lax.fori_loop(0, _NCHUNK, chunk_body, 0)


def _make_sc_pool():
    mesh = plsc.VectorSubcoreMesh(core_axis_name="c", subcore_axis_name="s")
    return functools.partial(
        pl.kernel,
        mesh=mesh,
        compiler_params=pltpu.CompilerParams(use_tc_tiling_on_sc=False),
        out_type=jax.ShapeDtypeStruct((_BATCH, _EMBED), jnp.float32),
        scratch_types=[
            pltpu.VMEM((_C, _G), jnp.int32),
            pltpu.VMEM((_GPH, _G, _EMBED), jnp.float32),
            pltpu.VMEM((_GPH, _G, _EMBED), jnp.float32),
            pltpu.VMEM((_C, _EMBED), jnp.float32),
            pltpu.VMEM_SHARED((_VOCAB, _EMBED), jnp.float32),
            pltpu.SemaphoreType.DMA,
            pltpu.SemaphoreType.DMA,
        ],
    )(_sc_pool)


def _mlp_body(sums_ref, w1_ref, b1_ref, w2r_ref, b2_ref, out_ref):
    pooled = sums_ref[...] * (1.0 / _MAXLEN)
    h = jnp.dot(pooled, w1_ref[...], preferred_element_type=jnp.float32)
    h = jnp.maximum(h + b1_ref[...], 0.0)
    z = jnp.sum(h * w2r_ref[...], axis=1, keepdims=True) + b2_ref[...]
    out_ref[...] = 1.0 / (1.0 + jnp.exp(-z))


def _mlp(sums, W1, b1, W2, b2):
    blk = 2048
    grid = (_BATCH // blk,)
    return pl.pallas_call(
        _mlp_body,
        grid=grid,
        in_specs=[
            pl.BlockSpec((blk, _EMBED), lambda i: (i, 0)),
            pl.BlockSpec((_EMBED, _EMBED), lambda i: (0, 0)),
            pl.BlockSpec((1, _EMBED), lambda i: (0, 0)),
            pl.BlockSpec((1, _EMBED), lambda i: (0, 0)),
            pl.BlockSpec((1, 1), lambda i: (0, 0)),
        ],
        out_specs=pl.BlockSpec((blk, 1), lambda i: (i, 0)),
        out_shape=jax.ShapeDtypeStruct((_BATCH, 1), jnp.float32),
    )(sums, W1, b1.reshape(1, _EMBED), W2.reshape(1, _EMBED), b2.reshape(1, 1))


def kernel(inputs, table, W1, b1, W2, b2):
    idx = inputs.astype(jnp.int32)
    sums = _make_sc_pool()(idx, table)
    return _mlp(sums, W1, b1, W2, b2)
